# baseline (speedup 1.0000x reference)
import functools

import jax
import jax.numpy as jnp
from jax.experimental import pallas as pl
from jax.experimental.pallas import tpu as pltpu
from jax.experimental.pallas import tpu_sc as plsc

BOARD_DIM = 42
VOCAB = 3
EMBED_DIM = 64
BATCH = 16384

NW = 32
BW_B = BATCH // NW
NG = BW_B // 16
XBLK = BW_B * BOARD_DIM
TC_PER_W = BW_B // 128
N_TR = BOARD_DIM * 8


def _combine_body(board_ref, pos_ref, out_ref):
    out_ref[...] = pos_ref[...][:, None, :] + board_ref[...][None, :, :]


def _build_combined(board_table, pos_table):
    return pl.pallas_call(
        _combine_body,
        out_shape=jax.ShapeDtypeStruct((BOARD_DIM, VOCAB, EMBED_DIM), jnp.float32),
    )(board_table, pos_table)


def _sc_body(x_hbm, comb_hbm, out_hbm, x_v, comb_v, xt_v, stage_v, osem):
    wid = jax.lax.axis_index("s") * 2 + jax.lax.axis_index("c")
    pltpu.sync_copy(comb_hbm, comb_v)
    pltpu.sync_copy(x_hbm.at[pl.ds(wid * BW_B, BW_B), :], x_v)
    lanes = jax.lax.iota(jnp.int32, 16)
    tc0 = wid * TC_PER_W

    def positions(p, _):
        for hh in range(2):
            dst = out_hbm.at[pl.ds(p * 8, 8), pl.ds(tc0 + hh * 2, 2), :, :]
            wait_prev = pltpu.make_async_copy(stage_v.at[hh], dst, osem.at[hh]).wait
            pl.when(p > 0)(wait_prev)

            def transpose_x(g, _, hh=hh):
                xv = plsc.load_gather(
                    x_v,
                    [(hh * 256 + g * 16) + lanes, jnp.broadcast_to(p, (16,))])
                xt_v[pl.ds(g * 16, 16)] = xv
                return 0

            jax.lax.fori_loop(0, 16, transpose_x, 0)
            rows = [comb_v[pl.ds(p * (VOCAB * EMBED_DIM) + v * EMBED_DIM + k * 16, 16)]
                    for v in range(VOCAB) for k in range(4)]

            def splat(v, d):
                row = rows[v * 4 + d // 16]
                idx = jnp.full((16, 1), d % 16, dtype=jnp.int32)
                return jax.lax.gather(
                    row, idx,
                    jax.lax.GatherDimensionNumbers(
                        offset_dims=(), collapsed_slice_dims=(0,),
                        start_index_map=(0,)),
                    (1,),
                    mode=jax.lax.GatherScatterMode.PROMISE_IN_BOUNDS)

            for r in range(8):
                cs = [[splat(v, dr * 8 + r) for v in range(VOCAB)]
                      for dr in range(8)]
                for dc in range(2):
                    def fill(gi, _, r=r, dc=dc, cs=cs, hh=hh):
                        xv = xt_v[pl.ds(dc * 128 + gi * 16, 16)]
                        m1 = xv == 1
                        m2 = xv == 2
                        for dr in range(8):
                            val = jax.lax.select_n(m1, cs[dr][0], cs[dr][1])
                            val = jax.lax.select_n(m2, val, cs[dr][2])
                            stage_v[hh, dr, dc, r, pl.ds(gi * 16, 16)] = val
                        return 0

                    jax.lax.fori_loop(0, 8, fill, 0)
            pltpu.async_copy(stage_v.at[hh], dst, osem.at[hh])
        return 0

    jax.lax.fori_loop(0, BOARD_DIM, positions, 0)

    for hh in range(2):
        dst = out_hbm.at[pl.ds((BOARD_DIM - 1) * 8, 8), pl.ds(tc0 + hh * 2, 2), :, :]
        pltpu.make_async_copy(stage_v.at[hh], dst, osem.at[hh]).wait()


@jax.jit
def kernel(x, board_table, pos_table):
    combined = _build_combined(board_table, pos_table).reshape(-1)
    x_in = x.astype(jnp.int32)

    mesh = plsc.VectorSubcoreMesh(core_axis_name="c", subcore_axis_name="s")
    out4 = pl.kernel(
        _sc_body,
        out_type=jax.ShapeDtypeStruct((N_TR, 128, 8, 128), jnp.float32),
        mesh=mesh,
        scratch_types=[
            pltpu.VMEM((BW_B, BOARD_DIM), jnp.int32),
            pltpu.VMEM((VOCAB * BOARD_DIM * EMBED_DIM,), jnp.float32),
            pltpu.VMEM((256,), jnp.int32),
            pltpu.VMEM((2, 8, 2, 8, 128), jnp.float32),
            pltpu.SemaphoreType.DMA((2,)),
        ],
        compiler_params=pltpu.CompilerParams(
            use_tc_tiling_on_sc=True, needs_layout_passes=False),
    )(x_in, combined)

    m = out4.transpose(0, 2, 1, 3).reshape(BOARD_DIM * EMBED_DIM, BATCH)
    m = m.reshape(BOARD_DIM, EMBED_DIM, BATCH)
    return jnp.transpose(m, (2, 0, 1))


# speedup vs baseline: 1.2579x; 1.2579x over previous
import jax
import jax.numpy as jnp
from jax.experimental import pallas as pl
from jax.experimental.pallas import tpu as pltpu
from jax.experimental.pallas import tpu_sc as plsc

BOARD_DIM = 42
VOCAB = 3
EMBED_DIM = 64
BATCH = 16384

NW = 32
BW_B = BATCH // NW
NG = BW_B // 16
XBLK = BW_B * BOARD_DIM
TC_PER_W = BW_B // 128
N_TR = BOARD_DIM * 8


def _combine_body(board_ref, pos_ref, out_ref):
    out_ref[...] = pos_ref[...][:, None, :] + board_ref[...][None, :, :]


def _build_combined(board_table, pos_table):
    return pl.pallas_call(
        _combine_body,
        out_shape=jax.ShapeDtypeStruct((BOARD_DIM, VOCAB, EMBED_DIM), jnp.float32),
    )(board_table, pos_table)


def _sc_body(x_hbm, comb_hbm, out_hbm, x_v, comb_v, xt_v, stage_v, osem):
    wid = jax.lax.axis_index("s") * 2 + jax.lax.axis_index("c")
    pltpu.sync_copy(comb_hbm, comb_v)
    pltpu.sync_copy(x_hbm.at[pl.ds(wid * XBLK, XBLK)], x_v)
    lanes = jax.lax.iota(jnp.int32, 16)
    xg_base = lanes * BOARD_DIM
    tc0 = wid * TC_PER_W

    def halves(t, _):
        for half in range(2):
            p = 2 * t + half
            dst = out_hbm.at[pl.ds(p * 8, 8), pl.ds(tc0, TC_PER_W), :, :]
            wait_prev = pltpu.make_async_copy(stage_v.at[half], dst, osem.at[half]).wait
            pl.when(t > 0)(wait_prev)

            def transpose_x(g, _):
                xv = plsc.load_gather(x_v, [xg_base + (g * (16 * BOARD_DIM) + p)])
                xt_v[pl.ds(g * 16, 16)] = xv
                return 0

            jax.lax.fori_loop(0, NG, transpose_x, 0)
            rows = [comb_v[pl.ds(p * (VOCAB * EMBED_DIM) + v * EMBED_DIM + k * 16, 16)]
                    for v in range(VOCAB) for k in range(4)]

            def splat(v, d):
                row = rows[v * 4 + d // 16]
                idx = jnp.full((16, 1), d % 16, dtype=jnp.int32)
                return jax.lax.gather(
                    row, idx,
                    jax.lax.GatherDimensionNumbers(
                        offset_dims=(), collapsed_slice_dims=(0,),
                        start_index_map=(0,)),
                    (1,),
                    mode=jax.lax.GatherScatterMode.PROMISE_IN_BOUNDS)

            for r in range(8):
                cs = [[splat(v, dr * 8 + r) for v in range(VOCAB)]
                      for dr in range(8)]
                for dc in range(TC_PER_W):
                    def fill(gi, _, r=r, dc=dc, cs=cs):
                        xv = xt_v[pl.ds(dc * 128 + gi * 16, 16)]
                        m1 = xv == 1
                        m2 = xv == 2
                        for dr in range(8):
                            val = jax.lax.select_n(m1, cs[dr][0], cs[dr][1])
                            val = jax.lax.select_n(m2, val, cs[dr][2])
                            stage_v[half, dr, dc, r, pl.ds(gi * 16, 16)] = val
                        return 0

                    jax.lax.fori_loop(0, NG // TC_PER_W, fill, 0)
            pltpu.async_copy(stage_v.at[half], dst, osem.at[half])
        return 0

    jax.lax.fori_loop(0, BOARD_DIM // 2, halves, 0)

    for half in range(2):
        p = BOARD_DIM - 2 + half
        dst = out_hbm.at[pl.ds(p * 8, 8), pl.ds(tc0, TC_PER_W), :, :]
        pltpu.make_async_copy(stage_v.at[half], dst, osem.at[half]).wait()


@jax.jit
def kernel(x, board_table, pos_table):
    combined = _build_combined(board_table, pos_table).reshape(-1)
    x_in = x.reshape(-1).astype(jnp.int32)

    mesh = plsc.VectorSubcoreMesh(core_axis_name="c", subcore_axis_name="s")
    out4 = pl.kernel(
        _sc_body,
        out_type=jax.ShapeDtypeStruct((N_TR, 128, 8, 128), jnp.float32),
        mesh=mesh,
        scratch_types=[
            pltpu.VMEM((XBLK,), jnp.int32),
            pltpu.VMEM((VOCAB * BOARD_DIM * EMBED_DIM,), jnp.float32),
            pltpu.VMEM((BW_B,), jnp.int32),
            pltpu.VMEM((2, 8, TC_PER_W, 8, 128), jnp.float32),
            pltpu.SemaphoreType.DMA((2,)),
        ],
        compiler_params=pltpu.CompilerParams(
            use_tc_tiling_on_sc=False, needs_layout_passes=False),
    )(x_in, combined)

    m = out4.transpose(0, 2, 1, 3).reshape(BOARD_DIM * EMBED_DIM, BATCH)
    m = m.reshape(BOARD_DIM, EMBED_DIM, BATCH)
    return jnp.transpose(m, (2, 0, 1))


# speedup vs baseline: 1.3667x; 1.0865x over previous
import jax
import jax.numpy as jnp
from jax.experimental import pallas as pl
from jax.experimental.pallas import tpu as pltpu
from jax.experimental.pallas import tpu_sc as plsc

BOARD_DIM = 42
VOCAB = 3
EMBED_DIM = 64
BATCH = 16384

NW = 32
BW_B = BATCH // NW
NG = BW_B // 16
XBLK = BW_B * BOARD_DIM
TC_PER_W = BW_B // 128
N_TR = BOARD_DIM * 8


def _combine_body(board_ref, pos_ref, x_ref, pt_ref, out_ref, xp_ref):
    out_ref[...] = pos_ref[...][:, None, :] + board_ref[...][None, :, :]
    xf = x_ref[...].astype(jnp.float32)
    packed = jax.lax.dot_general(pt_ref[...], xf, (((1,), (0,)), ((), ())))
    xp_ref[...] = packed.astype(jnp.int32).reshape(6, 128, 128)


def _build_combined(board_table, pos_table, x, pack_t):
    return pl.pallas_call(
        _combine_body,
        out_shape=(
            jax.ShapeDtypeStruct((BOARD_DIM, VOCAB, EMBED_DIM), jnp.float32),
            jax.ShapeDtypeStruct((6, 128, 128), jnp.int32),
        ),
    )(board_table, pos_table, x, pack_t)


def _sc_body(x_hbm, comb_hbm, out_hbm, x_v, comb_v, xt_v, stage_v, osem):
    wid = jax.lax.axis_index("s") * 2 + jax.lax.axis_index("c")
    pltpu.sync_copy(comb_hbm, comb_v)
    for s in range(6):
        pltpu.sync_copy(x_hbm.at[pl.ds(s * BATCH + wid * BW_B, BW_B)],
                        x_v.at[pl.ds(s * BW_B, BW_B)])
    tc0 = wid * TC_PER_W

    def halves(t, _):
        for half in range(2):
            p = 2 * t + half
            sw = (p // 8) * BW_B
            sh = 2 * (p % 8)
            dst = out_hbm.at[pl.ds(p * 8, 8), pl.ds(tc0, TC_PER_W), :, :]
            wait_prev = pltpu.make_async_copy(stage_v.at[half], dst, osem.at[half]).wait
            pl.when(t > 0)(wait_prev)

            def transpose_x(g, _):
                pk = x_v[pl.ds(sw + g * 16, 16)]
                xt_v[pl.ds(g * 16, 16)] = (pk >> sh) & 3
                return 0

            jax.lax.fori_loop(0, NG, transpose_x, 0)
            rows = [comb_v[pl.ds(p * (VOCAB * EMBED_DIM) + v * EMBED_DIM + k * 16, 16)]
                    for v in range(VOCAB) for k in range(4)]

            def splat(v, d):
                row = rows[v * 4 + d // 16]
                idx = jnp.full((16, 1), d % 16, dtype=jnp.int32)
                return jax.lax.gather(
                    row, idx,
                    jax.lax.GatherDimensionNumbers(
                        offset_dims=(), collapsed_slice_dims=(0,),
                        start_index_map=(0,)),
                    (1,),
                    mode=jax.lax.GatherScatterMode.PROMISE_IN_BOUNDS)

            for r in range(8):
                cs = [[splat(v, dr * 8 + r) for v in range(VOCAB)]
                      for dr in range(8)]
                for dc in range(TC_PER_W):
                    def fill(gi, _, r=r, dc=dc, cs=cs):
                        xv = xt_v[pl.ds(dc * 128 + gi * 16, 16)]
                        m1 = xv == 1
                        m2 = xv == 2
                        for dr in range(8):
                            val = jax.lax.select_n(m1, cs[dr][0], cs[dr][1])
                            val = jax.lax.select_n(m2, val, cs[dr][2])
                            stage_v[half, dr, dc, r, pl.ds(gi * 16, 16)] = val
                        return 0

                    jax.lax.fori_loop(0, NG // TC_PER_W, fill, 0)
            pltpu.async_copy(stage_v.at[half], dst, osem.at[half])
        return 0

    jax.lax.fori_loop(0, BOARD_DIM // 2, halves, 0)

    for half in range(2):
        p = BOARD_DIM - 2 + half
        dst = out_hbm.at[pl.ds(p * 8, 8), pl.ds(tc0, TC_PER_W), :, :]
        pltpu.make_async_copy(stage_v.at[half], dst, osem.at[half]).wait()


@jax.jit
def kernel(x, board_table, pos_table):
    pcol = jnp.arange(BOARD_DIM)
    pack_t = jnp.where(pcol[None, :] // 8 == jnp.arange(6)[:, None],
                       (4.0 ** (pcol % 8))[None, :], 0.0).astype(jnp.float32)
    combined, xp = _build_combined(
        board_table, pos_table, x.astype(jnp.int32).T, pack_t)
    combined = combined.reshape(-1)
    xp_flat = xp.reshape(-1)

    mesh = plsc.VectorSubcoreMesh(core_axis_name="c", subcore_axis_name="s")
    out4 = pl.kernel(
        _sc_body,
        out_type=jax.ShapeDtypeStruct((N_TR, 128, 8, 128), jnp.float32),
        mesh=mesh,
        scratch_types=[
            pltpu.VMEM((6 * BW_B,), jnp.int32),
            pltpu.VMEM((VOCAB * BOARD_DIM * EMBED_DIM,), jnp.float32),
            pltpu.VMEM((BW_B,), jnp.int32),
            pltpu.VMEM((2, 8, TC_PER_W, 8, 128), jnp.float32),
            pltpu.SemaphoreType.DMA((2,)),
        ],
        compiler_params=pltpu.CompilerParams(
            use_tc_tiling_on_sc=False, needs_layout_passes=False),
    )(xp_flat, combined)

    m = out4.transpose(0, 2, 1, 3).reshape(BOARD_DIM * EMBED_DIM, BATCH)
    m = m.reshape(BOARD_DIM, EMBED_DIM, BATCH)
    return jnp.transpose(m, (2, 0, 1))
